# baseline (speedup 1.0000x reference)
import jax
import jax.numpy as jnp
import numpy as np
from jax.experimental import pallas as pl

_B = 16384
_C = 40
_BAL_POS = 0.3 * _B
_BAL_NEG = _B - _BAL_POS
_EDGE_LO = 27.0 / 30.0
_EDGE_HI = 1.0 + 1e-6
_HI_BITS = 0x3F800000


def _bce(x, t):
    return jnp.maximum(x, 0.0) - x * t + jnp.log1p(jnp.exp(-jnp.abs(x)))


def _tf2x32(k1, k2, x1, x2):
    def rotl(x, d):
        return ((x << np.uint32(d)) | (x >> np.uint32(32 - d))).astype(np.uint32)
    rot = [(13, 15, 26, 6), (17, 29, 16, 24)]
    ks = [np.uint32(k1), np.uint32(k2),
          np.uint32(k1) ^ np.uint32(k2) ^ np.uint32(0x1BD11BDA)]
    x = [x1.astype(np.uint32) + ks[0], x2.astype(np.uint32) + ks[1]]
    for i in range(5):
        for r in rot[i % 2]:
            x[0] = (x[0] + x[1]).astype(np.uint32)
            x[1] = x[0] ^ rotl(x[1], r)
        x[0] = (x[0] + ks[(i + 1) % 3]).astype(np.uint32)
        x[1] = (x[1] + ks[(i + 2) % 3] + np.uint32(i + 1)).astype(np.uint32)
    return x[0], x[1]


def _np_uniform(key, shape):
    n = int(np.prod(shape))
    idx = np.arange(n, dtype=np.uint64)
    hi = (idx >> np.uint64(32)).astype(np.uint32)
    lo = idx.astype(np.uint32)
    b1, b2 = _tf2x32(key[0], key[1], hi, lo)
    bits = b1 ^ b2
    u = ((bits >> np.uint32(9)) | np.uint32(0x3F800000)).view(np.float32) \
        - np.float32(1.0)
    return np.maximum(np.float32(0.0), u).reshape(shape)


def _rng_consts():
    b1, b2 = _tf2x32(np.uint32(0), np.uint32(42),
                     np.zeros(2, np.uint32), np.arange(2, dtype=np.uint32))
    kr1, kr2 = (b1[0], b2[0]), (b1[1], b2[1])
    rand_mat = _np_uniform(kr1, (_B, _C))
    urand = _np_uniform(kr2, (_C,))
    return np.ascontiguousarray(rand_mat.T), urand.reshape(_C, 1)


_RAND_T, _URAND = _rng_consts()


def _body(pred_ref, tgt_ref, rand_ref, ur_ref, out_ref):
    pred = pred_ref[...].T
    tgt = tgt_ref[...].T
    bce = _bce(pred, tgt)
    g = jnp.abs(jax.nn.sigmoid(pred) - tgt)

    loss_col = jnp.sum(bce, axis=1, keepdims=True)
    pos_sum = jnp.sum(tgt, axis=1, keepdims=True)

    ln_loss = jnp.log10(1.0 + loss_col)
    mn = jnp.min(ln_loss)
    mx = jnp.max(ln_loss)
    norm_loss = 5.0 - 10.0 * (ln_loss - mn) / (mx - mn)
    s = jax.nn.sigmoid(norm_loss)
    dropout_rate = jnp.where(s > 0.0, s, 0.0)

    neg_sum = _B - pos_sum
    pos_gt = pos_sum > _BAL_POS
    neg_gt = neg_sum > _BAL_NEG
    balance_num = jnp.where(pos_gt, _BAL_POS, 0.0)
    balance_num = jnp.where(neg_gt, _BAL_NEG, balance_num)
    dnum = jnp.where(pos_gt, pos_sum - _BAL_POS, 0.0)
    dnum = jnp.where(neg_gt, neg_sum - _BAL_NEG, dnum)
    k = dnum.astype(jnp.int32)

    maj_label = pos_gt.astype(jnp.float32)
    maj = tgt == maj_label
    maj_count = jnp.where(pos_gt, pos_sum, neg_sum)
    min_count = _B - maj_count
    hf_col = balance_num / jnp.maximum(maj_count, 1.0)
    mf_col = jnp.where(min_count > 0.0,
                       (_B - balance_num) / jnp.maximum(min_count, 1.0),
                       1.0)

    gbits = jax.lax.bitcast_convert_type(g, jnp.int32)
    gb = jnp.where(maj, gbits, jnp.int32(0x7F000000))

    def s1(_, c):
        lo, hi = c
        mid = jax.lax.shift_right_logical(lo + hi, 1)
        cnt = jnp.count_nonzero(gb <= mid, axis=1, keepdims=True).astype(jnp.int32)
        ge = cnt >= k
        return jnp.where(ge, lo, mid + 1), jnp.where(ge, mid, hi)

    _, v = jax.lax.fori_loop(
        0, 30, s1,
        (jnp.zeros((_C, 1), jnp.int32), jnp.full((_C, 1), _HI_BITS, jnp.int32)))

    ridx = jax.lax.broadcasted_iota(jnp.int32, (_C, _B), 1)
    key2 = jnp.where(gb < v, jnp.int32(-1),
                     jnp.where(gb == v, ridx, jnp.int32(1 << 30)))

    def s2(_, c):
        lo, hi = c
        mid = jax.lax.shift_right_logical(lo + hi, 1)
        cnt = jnp.count_nonzero(key2 < mid, axis=1, keepdims=True).astype(jnp.int32)
        ge = cnt >= k
        return jnp.where(ge, lo, mid + 1), jnp.where(ge, mid, hi)

    _, t = jax.lax.fori_loop(
        0, 15, s2,
        (jnp.zeros((_C, 1), jnp.int32), jnp.full((_C, 1), _B, jnp.int32)))

    drop = key2 < t

    easy_w = jnp.where(drop, 0.0, 1.0) * jnp.where(maj, 1.0, mf_col)
    hard_mask = ur_ref[...] > dropout_rate
    hard_w = jnp.where(maj, hf_col, 1.0)
    weights = jnp.where(hard_mask, hard_w, easy_w)
    idxs = (g >= _EDGE_LO) & (g < _EDGE_HI)
    drop_idxs = (rand_ref[...] > dropout_rate).astype(jnp.float32)
    weights = weights * (1.0 - drop_idxs * idxs.astype(jnp.float32))
    per_row = jnp.sum(bce * weights, axis=1, keepdims=True)
    out_ref[...] = jnp.sum(per_row, axis=0, keepdims=True) / (_B * _C)


def kernel(pred, target):
    out = pl.pallas_call(
        _body,
        out_shape=jax.ShapeDtypeStruct((1, 1), jnp.float32),
    )(pred, target, _RAND_T, _URAND)
    return out[0, 0]


# speedup vs baseline: 1.6247x; 1.6247x over previous
import jax
import jax.numpy as jnp
import numpy as np
from jax.experimental import pallas as pl

_B = 16384
_C = 40
_BAL_POS = 0.3 * _B
_BAL_NEG = _B - _BAL_POS
_EDGE_LO = 27.0 / 30.0
_EDGE_HI = 1.0 + 1e-6
_HI_BITS = 0x3F800000


def _bce(x, t):
    return jnp.maximum(x, 0.0) - x * t + jnp.log1p(jnp.exp(-jnp.abs(x)))


def _tf2x32(k1, k2, x1, x2):
    def rotl(x, d):
        return ((x << np.uint32(d)) | (x >> np.uint32(32 - d))).astype(np.uint32)
    rot = [(13, 15, 26, 6), (17, 29, 16, 24)]
    ks = [np.uint32(k1), np.uint32(k2),
          np.uint32(k1) ^ np.uint32(k2) ^ np.uint32(0x1BD11BDA)]
    x = [x1.astype(np.uint32) + ks[0], x2.astype(np.uint32) + ks[1]]
    for i in range(5):
        for r in rot[i % 2]:
            x[0] = (x[0] + x[1]).astype(np.uint32)
            x[1] = x[0] ^ rotl(x[1], r)
        x[0] = (x[0] + ks[(i + 1) % 3]).astype(np.uint32)
        x[1] = (x[1] + ks[(i + 2) % 3] + np.uint32(i + 1)).astype(np.uint32)
    return x[0], x[1]


def _np_uniform(key, shape):
    n = int(np.prod(shape))
    idx = np.arange(n, dtype=np.uint64)
    hi = (idx >> np.uint64(32)).astype(np.uint32)
    lo = idx.astype(np.uint32)
    b1, b2 = _tf2x32(key[0], key[1], hi, lo)
    bits = b1 ^ b2
    u = ((bits >> np.uint32(9)) | np.uint32(0x3F800000)).view(np.float32) \
        - np.float32(1.0)
    return np.maximum(np.float32(0.0), u).reshape(shape)


def _rng_consts():
    b1, b2 = _tf2x32(np.uint32(0), np.uint32(42),
                     np.zeros(2, np.uint32), np.arange(2, dtype=np.uint32))
    kr1, kr2 = (b1[0], b2[0]), (b1[1], b2[1])
    rand_mat = _np_uniform(kr1, (_B, _C))
    urand = _np_uniform(kr2, (_C,))
    return np.ascontiguousarray(rand_mat.T), urand.reshape(_C, 1)


_RAND_T, _URAND = _rng_consts()


def _body(pred_ref, tgt_ref, rand_ref, ur_ref, out_ref):
    pred = pred_ref[...]
    tgt = tgt_ref[...]
    bce = _bce(pred, tgt)
    g = jnp.abs(jax.nn.sigmoid(pred) - tgt)

    loss_col = jnp.sum(bce, axis=1, keepdims=True)
    pos_sum = jnp.sum(tgt, axis=1, keepdims=True)

    ln_loss = jnp.log10(1.0 + loss_col)
    mn = jnp.min(ln_loss)
    mx = jnp.max(ln_loss)
    norm_loss = 5.0 - 10.0 * (ln_loss - mn) / (mx - mn)
    s = jax.nn.sigmoid(norm_loss)
    dropout_rate = jnp.where(s > 0.0, s, 0.0)

    neg_sum = _B - pos_sum
    pos_gt = pos_sum > _BAL_POS
    neg_gt = neg_sum > _BAL_NEG
    balance_num = jnp.where(pos_gt, _BAL_POS, 0.0)
    balance_num = jnp.where(neg_gt, _BAL_NEG, balance_num)
    dnum = jnp.where(pos_gt, pos_sum - _BAL_POS, 0.0)
    dnum = jnp.where(neg_gt, neg_sum - _BAL_NEG, dnum)
    k = dnum.astype(jnp.int32)

    maj_label = pos_gt.astype(jnp.float32)
    maj = tgt == maj_label
    maj_count = jnp.where(pos_gt, pos_sum, neg_sum)
    min_count = _B - maj_count
    hf_col = balance_num / jnp.maximum(maj_count, 1.0)
    mf_col = jnp.where(min_count > 0.0,
                       (_B - balance_num) / jnp.maximum(min_count, 1.0),
                       1.0)

    gbits = jax.lax.bitcast_convert_type(g, jnp.int32)
    gb = jnp.where(maj, gbits, jnp.int32(0x3FFF8000))
    hi15 = gb >> 15
    klo = gb & 0x7FFF
    _H = _B // 2

    def _pack(x):
        p = x[:, :_H] | (x[:, _H:] << 16)
        return p | jnp.int32(0x80008000 - (1 << 32))

    def _cge(p, theta):
        d = p - theta * jnp.int32(0x00010001)
        s = jnp.sum((d >> 15) & jnp.int32(0x00010001), axis=1, keepdims=True)
        return (s & 0xFFFF) + jax.lax.shift_right_logical(s, 16)

    def _sel_search(p, kk, hi0):
        def step(_, c):
            lo, hi = c
            mid = (lo + hi) >> 1
            ge = _B - _cge(p, mid + 1) >= kk
            return jnp.where(ge, lo, mid + 1), jnp.where(ge, mid, hi)
        return jax.lax.fori_loop(
            0, 15, step,
            (jnp.zeros((_C, 1), jnp.int32),
             jnp.full((_C, 1), hi0, jnp.int32)))[1]

    p1 = _pack(hi15)
    vh = _sel_search(p1, k, 32513)

    above2 = _cge(p1, vh + 1)
    k2el = jnp.where(hi15 == vh, klo, 0)
    vl = _sel_search(_pack(k2el), k + above2, 32767)

    ridx = jax.lax.broadcasted_iota(jnp.int32, (_C, _B), 1)
    tie = (hi15 == vh) & (klo == vl)
    below = (hi15 < vh) | ((hi15 == vh) & (klo < vl))
    k3el = jnp.where(tie, ridx, jnp.where(below, 0, jnp.int32(0x7FFF)))
    p3 = _pack(k3el)

    def s3(_, c):
        lo, hi = c
        mid = (lo + hi) >> 1
        ge = _B - _cge(p3, mid) >= k
        return jnp.where(ge, lo, mid + 1), jnp.where(ge, mid, hi)

    t = jax.lax.fori_loop(
        0, 15, s3,
        (jnp.zeros((_C, 1), jnp.int32),
         jnp.full((_C, 1), _B, jnp.int32)))[1]

    drop = below | (tie & (ridx < t))

    easy_w = jnp.where(drop, 0.0, 1.0) * jnp.where(maj, 1.0, mf_col)
    hard_mask = ur_ref[...] > dropout_rate
    hard_w = jnp.where(maj, hf_col, 1.0)
    weights = jnp.where(hard_mask, hard_w, easy_w)
    idxs = (g >= _EDGE_LO) & (g < _EDGE_HI)
    drop_idxs = (rand_ref[...] > dropout_rate).astype(jnp.float32)
    weights = weights * (1.0 - drop_idxs * idxs.astype(jnp.float32))
    per_row = jnp.sum(bce * weights, axis=1, keepdims=True)
    out_ref[...] = jnp.sum(per_row, axis=0, keepdims=True) / (_B * _C)


def kernel(pred, target):
    out = pl.pallas_call(
        _body,
        out_shape=jax.ShapeDtypeStruct((1, 1), jnp.float32),
    )(pred.T, target.T, _RAND_T, _URAND)
    return out[0, 0]
